# Initial kernel scaffold; baseline (speedup 1.0000x reference)
import jax
import jax.numpy as jnp
from jax.experimental import pallas as pl


def kernel(attn):
    raise NotImplementedError("write your pallas kernel here")


# speedup vs baseline: 22.0145x; 22.0145x over previous
import functools

import jax
import jax.numpy as jnp
from jax import lax
from jax.experimental import pallas as pl
from jax.experimental.pallas import tpu as pltpu
from jax.experimental.pallas import tpu_sc as plsc

B = 16
S = 2048
L16 = 16
PAD = S + L16
NROW = S // 2


def _row_accumulate(i, seg, acc):
    nfull = (S - i) // L16

    def body(t, carry):
        off = t * L16
        acc[pl.ds(off, L16)] = acc[pl.ds(off, L16)] + seg[pl.ds(i + off, L16)]
        return carry

    lax.fori_loop(0, nfull, body, 0)

    base = nfull * L16
    rem = (S - i) - base
    lanes = jax.lax.iota(jnp.int32, L16)
    v = seg[pl.ds(i + base, L16)]
    v = jnp.where(lanes < rem, v, jnp.zeros((L16,), jnp.float32))
    acc[pl.ds(base, L16)] = acc[pl.ds(base, L16)] + v


def _make_sc_kernel():
    mesh = plsc.VectorSubcoreMesh(core_axis_name="c", subcore_axis_name="s")

    @functools.partial(
        pl.kernel,
        out_type=jax.ShapeDtypeStruct((B, S), jnp.float32),
        mesh=mesh,
        scratch_types=[
            pltpu.VMEM((PAD,), jnp.float32),
            pltpu.VMEM((PAD,), jnp.float32),
            pltpu.VMEM((PAD,), jnp.float32),
            pltpu.VMEM_SHARED((16, S), jnp.float32),
            pltpu.VMEM((S // 2,), jnp.float32),
            pltpu.VMEM((S // 2,), jnp.float32),
            pltpu.VMEM((S // 2,), jnp.float32),
            pltpu.SemaphoreType.DMA,
            pltpu.SemaphoreType.DMA,
        ],
    )
    def diag_mean(attn, out, seg0, seg1, acc, shared, pa, pb, res, sem0, sem1):
        c = lax.axis_index("c")
        s = lax.axis_index("s")
        batch = c * 8 + s // 2
        half = s % 2

        def zbody(t, carry):
            acc[pl.ds(t * L16, L16)] = jnp.zeros((L16,), jnp.float32)
            return carry

        lax.fori_loop(0, PAD // L16, zbody, 0)

        def row_of(r):
            return 2 * r + half

        def start(r, seg, sem):
            pltpu.async_copy(attn.at[batch, row_of(r)], seg.at[pl.ds(0, S)], sem)

        def wait(seg, sem):
            pltpu.make_async_copy(attn.at[batch, 0], seg.at[pl.ds(0, S)], sem).wait()

        start(0, seg0, sem0)
        start(1, seg1, sem1)

        def main(rp, carry):
            r0 = 2 * rp
            wait(seg0, sem0)
            _row_accumulate(row_of(r0), seg0, acc)
            start(r0 + 2, seg0, sem0)
            wait(seg1, sem1)
            _row_accumulate(row_of(r0 + 1), seg1, acc)
            start(r0 + 3, seg1, sem1)
            return carry

        lax.fori_loop(0, NROW // 2 - 1, main, 0)

        wait(seg0, sem0)
        _row_accumulate(row_of(NROW - 2), seg0, acc)
        wait(seg1, sem1)
        _row_accumulate(row_of(NROW - 1), seg1, acc)

        pltpu.sync_copy(acc.at[pl.ds(0, S)], shared.at[s])
        plsc.subcore_barrier()

        s0 = (s // 2) * 2
        off = (s % 2) * (S // 2)
        pltpu.sync_copy(shared.at[s0, pl.ds(off, S // 2)], pa)
        pltpu.sync_copy(shared.at[s0 + 1, pl.ds(off, S // 2)], pb)

        lanes = jax.lax.iota(jnp.int32, L16)

        def dbody(t, carry):
            o = t * L16
            d = off + o + lanes
            cnt = (S - d).astype(jnp.float32)
            res[pl.ds(o, L16)] = (pa[pl.ds(o, L16)] + pb[pl.ds(o, L16)]) / cnt
            return carry

        lax.fori_loop(0, (S // 2) // L16, dbody, 0)

        pltpu.sync_copy(res, out.at[batch, pl.ds(off, S // 2)])

    return diag_mean


_diag_mean_sc = _make_sc_kernel()


@jax.jit
def kernel(attn):
    return _diag_mean_sc(attn)


# speedup vs baseline: 23.0937x; 1.0490x over previous
import functools

import jax
import jax.numpy as jnp
from jax import lax
from jax.experimental import pallas as pl
from jax.experimental.pallas import tpu as pltpu
from jax.experimental.pallas import tpu_sc as plsc

B = 16
S = 2048
L16 = 16
UNROLL = 8
GRP = UNROLL * L16
PAD = S + GRP
NROW = S // 2


def _row_accumulate(i, seg, acc):
    L = S - i
    ngrp = L // GRP

    def body(g, carry):
        off = g * GRP
        for u in range(UNROLL):
            o = off + u * L16
            acc[pl.ds(o, L16)] = acc[pl.ds(o, L16)] + seg[pl.ds(i + o, L16)]
        return carry

    lax.fori_loop(0, ngrp, body, 0)

    base = ngrp * GRP
    lanes = jax.lax.iota(jnp.int32, L16)
    zero = jnp.zeros((L16,), jnp.float32)
    for u in range(UNROLL):
        o = base + u * L16
        v = seg[pl.ds(i + o, L16)]
        v = jnp.where(lanes < (L - o), v, zero)
        acc[pl.ds(o, L16)] = acc[pl.ds(o, L16)] + v


def _make_sc_kernel():
    mesh = plsc.VectorSubcoreMesh(core_axis_name="c", subcore_axis_name="s")

    @functools.partial(
        pl.kernel,
        out_type=jax.ShapeDtypeStruct((B, S), jnp.float32),
        mesh=mesh,
        scratch_types=[
            pltpu.VMEM((PAD,), jnp.float32),
            pltpu.VMEM((PAD,), jnp.float32),
            pltpu.VMEM((PAD,), jnp.float32),
            pltpu.VMEM_SHARED((16, S), jnp.float32),
            pltpu.VMEM((S // 2,), jnp.float32),
            pltpu.VMEM((S // 2,), jnp.float32),
            pltpu.VMEM((S // 2,), jnp.float32),
            pltpu.SemaphoreType.DMA,
            pltpu.SemaphoreType.DMA,
        ],
    )
    def diag_mean(attn, out, seg0, seg1, acc, shared, pa, pb, res, sem0, sem1):
        c = lax.axis_index("c")
        s = lax.axis_index("s")
        batch = c * 8 + s // 2
        half = s % 2

        def zbody(t, carry):
            acc[pl.ds(t * L16, L16)] = jnp.zeros((L16,), jnp.float32)
            return carry

        lax.fori_loop(0, PAD // L16, zbody, 0)

        def row_of(r):
            return 2 * r + half

        def start(r, seg, sem):
            pltpu.async_copy(attn.at[batch, row_of(r)], seg.at[pl.ds(0, S)], sem)

        def wait(seg, sem):
            pltpu.make_async_copy(attn.at[batch, 0], seg.at[pl.ds(0, S)], sem).wait()

        start(0, seg0, sem0)
        start(1, seg1, sem1)

        def main(rp, carry):
            r0 = 2 * rp
            wait(seg0, sem0)
            _row_accumulate(row_of(r0), seg0, acc)
            start(r0 + 2, seg0, sem0)
            wait(seg1, sem1)
            _row_accumulate(row_of(r0 + 1), seg1, acc)
            start(r0 + 3, seg1, sem1)
            return carry

        lax.fori_loop(0, NROW // 2 - 1, main, 0)

        wait(seg0, sem0)
        _row_accumulate(row_of(NROW - 2), seg0, acc)
        wait(seg1, sem1)
        _row_accumulate(row_of(NROW - 1), seg1, acc)

        pltpu.sync_copy(acc.at[pl.ds(0, S)], shared.at[s])
        plsc.subcore_barrier()

        s0 = (s // 2) * 2
        off = (s % 2) * (S // 2)
        pltpu.sync_copy(shared.at[s0, pl.ds(off, S // 2)], pa)
        pltpu.sync_copy(shared.at[s0 + 1, pl.ds(off, S // 2)], pb)

        lanes = jax.lax.iota(jnp.int32, L16)

        def dbody(t, carry):
            o = t * L16
            d = off + o + lanes
            cnt = (S - d).astype(jnp.float32)
            res[pl.ds(o, L16)] = (pa[pl.ds(o, L16)] + pb[pl.ds(o, L16)]) / cnt
            return carry

        lax.fori_loop(0, (S // 2) // L16, dbody, 0)

        pltpu.sync_copy(res, out.at[batch, pl.ds(off, S // 2)])

    return diag_mean


_diag_mean_sc = _make_sc_kernel()


@jax.jit
def kernel(attn):
    return _diag_mean_sc(attn)


# speedup vs baseline: 32.3023x; 1.3987x over previous
import functools

import jax
import jax.numpy as jnp
from jax import lax
from jax.experimental import pallas as pl
from jax.experimental.pallas import tpu as pltpu
from jax.experimental.pallas import tpu_sc as plsc

B = 16
S = 2048
L16 = 16
UNROLL = 8
GRP = UNROLL * L16
PAD = S + GRP
NROW = S // 2
NBUF = 8


def _row_accumulate(i, seg, acc):
    L = S - i
    ngrp = L // GRP

    def body(g, carry):
        off = g * GRP
        for u in range(UNROLL):
            o = off + u * L16
            acc[pl.ds(o, L16)] = acc[pl.ds(o, L16)] + seg[pl.ds(i + o, L16)]
        return carry

    lax.fori_loop(0, ngrp, body, 0)

    base = ngrp * GRP
    lanes = jax.lax.iota(jnp.int32, L16)
    zero = jnp.zeros((L16,), jnp.float32)
    for u in range(UNROLL):
        o = base + u * L16
        v = seg[pl.ds(i + o, L16)]
        v = jnp.where(lanes < (L - o), v, zero)
        acc[pl.ds(o, L16)] = acc[pl.ds(o, L16)] + v


def _make_sc_kernel():
    mesh = plsc.VectorSubcoreMesh(core_axis_name="c", subcore_axis_name="s")

    @functools.partial(
        pl.kernel,
        out_type=jax.ShapeDtypeStruct((B, S), jnp.float32),
        mesh=mesh,
        scratch_types=(
            [pltpu.VMEM((PAD,), jnp.float32) for _ in range(NBUF)]
            + [
                pltpu.VMEM((PAD,), jnp.float32),
                pltpu.VMEM_SHARED((16, S), jnp.float32),
                pltpu.VMEM((S // 2,), jnp.float32),
                pltpu.VMEM((S // 2,), jnp.float32),
                pltpu.VMEM((S // 2,), jnp.float32),
            ]
            + [pltpu.SemaphoreType.DMA for _ in range(NBUF)]
        ),
    )
    def diag_mean(attn, out, *refs):
        segs = refs[:NBUF]
        acc, shared, pa, pb, res = refs[NBUF:NBUF + 5]
        sems = refs[NBUF + 5:]
        c = lax.axis_index("c")
        s = lax.axis_index("s")
        batch = c * 8 + s // 2
        half = s % 2

        def zbody(t, carry):
            acc[pl.ds(t * L16, L16)] = jnp.zeros((L16,), jnp.float32)
            return carry

        lax.fori_loop(0, PAD // L16, zbody, 0)

        def row_of(r):
            return 2 * r + half

        def start(r, seg, sem):
            pltpu.async_copy(attn.at[batch, row_of(r)], seg.at[pl.ds(0, S)], sem)

        def wait(seg, sem):
            pltpu.make_async_copy(attn.at[batch, 0], seg.at[pl.ds(0, S)], sem).wait()

        for u in range(NBUF):
            start(u, segs[u], sems[u])

        def main(rp, carry):
            r0 = rp * NBUF
            for u in range(NBUF):
                wait(segs[u], sems[u])
                _row_accumulate(row_of(r0 + u), segs[u], acc)
                start(r0 + u + NBUF, segs[u], sems[u])
            return carry

        lax.fori_loop(0, NROW // NBUF - 1, main, 0)

        for u in range(NBUF):
            wait(segs[u], sems[u])
            _row_accumulate(NROW * 2 - 2 * NBUF + 2 * u + half, segs[u], acc)

        pltpu.sync_copy(acc.at[pl.ds(0, S)], shared.at[s])
        plsc.subcore_barrier()

        s0 = (s // 2) * 2
        off = (s % 2) * (S // 2)
        pltpu.sync_copy(shared.at[s0, pl.ds(off, S // 2)], pa)
        pltpu.sync_copy(shared.at[s0 + 1, pl.ds(off, S // 2)], pb)

        lanes = jax.lax.iota(jnp.int32, L16)

        def dbody(t, carry):
            o = t * L16
            d = off + o + lanes
            cnt = (S - d).astype(jnp.float32)
            res[pl.ds(o, L16)] = (pa[pl.ds(o, L16)] + pb[pl.ds(o, L16)]) / cnt
            return carry

        lax.fori_loop(0, (S // 2) // L16, dbody, 0)

        pltpu.sync_copy(res, out.at[batch, pl.ds(off, S // 2)])

    return diag_mean


_diag_mean_sc = _make_sc_kernel()


@jax.jit
def kernel(attn):
    return _diag_mean_sc(attn)
